# Initial kernel scaffold; baseline (speedup 1.0000x reference)
import jax
import jax.numpy as jnp
from jax.experimental import pallas as pl


def kernel(x_global, x_lesion, x_cause, global_txt, text_mask, ei_g2l, ei_l2g, ei_l2l, ei_l2c, ei_c2l, ei_g2c, ei_c2g, cause_batch, params):
    raise NotImplementedError("write your pallas kernel here")


# speedup vs baseline: 1.0663x; 1.0663x over previous
import functools

import jax
import jax.numpy as jnp
from jax.experimental import pallas as pl
from jax.experimental.pallas import tpu as pltpu

D = 512
H = 256
_INTERPRET = False


def _ln(x, g, b, eps=1e-5):
    m = x.mean(-1, keepdims=True)
    v = ((x - m) ** 2).mean(-1, keepdims=True)
    return (x - m) / jnp.sqrt(v + eps) * g + b


def _gelu(x):
    return x * 0.5 * (1.0 + jax.lax.erf(x * (2.0 ** -0.5)))


def _full2d(a):
    return pl.BlockSpec(a.shape, lambda i: (0, 0))


def _fuse_proj_body(xg_ref, gt_ref, tm_ref, gate_ref, wfg_ref, wft_ref, bf_ref,
                    w1_ref, b1_ref, g1_ref, n1_ref, w2_ref, b2_ref, g2_ref,
                    n2_ref, out_ref):
    xg = xg_ref[...]
    fused = _gelu(xg @ wfg_ref[...] + gt_ref[...] @ wft_ref[...] + bf_ref[...])
    g = xg + gate_ref[0, 0] * (tm_ref[...] * fused)
    h = g @ w1_ref[...] + b1_ref[...]
    h = _gelu(_ln(h, g1_ref[...], n1_ref[...]))
    h = h @ w2_ref[...] + b2_ref[...]
    h = _ln(h, g2_ref[...], n2_ref[...])
    out_ref[0, ...] = h[:, :128]
    out_ref[1, ...] = h[:, 128:]


def _fuse_proj(xg, gt, tm, gate, fuse_p, proj_p, bm):
    n = xg.shape[0]
    wfg = fuse_p['W'][:D]
    wft = fuse_p['W'][D:]
    args = (xg, gt, tm.reshape(n, 1), gate.reshape(1, 1), wfg, wft,
            fuse_p['b'].reshape(1, D),
            proj_p['W1'], proj_p['b1'].reshape(1, H),
            proj_p['g1'].reshape(1, H), proj_p['bn1'].reshape(1, H),
            proj_p['W2'], proj_p['b2'].reshape(1, H),
            proj_p['g2'].reshape(1, H), proj_p['bn2'].reshape(1, H))
    in_specs = [
        pl.BlockSpec((bm, D), lambda i: (i, 0)),
        pl.BlockSpec((bm, D), lambda i: (i, 0)),
        pl.BlockSpec((bm, 1), lambda i: (i, 0)),
        pl.BlockSpec((1, 1), lambda i: (0, 0)),
    ] + [_full2d(a) for a in args[4:]]
    return pl.pallas_call(
        _fuse_proj_body,
        grid=(n // bm,),
        in_specs=in_specs,
        out_specs=pl.BlockSpec((2, bm, 128), lambda i: (0, i, 0)),
        out_shape=jax.ShapeDtypeStruct((2, n, 128), jnp.float32),
        interpret=_INTERPRET,
    )(*args)


def _proj_body(x_ref, w1_ref, b1_ref, g1_ref, n1_ref, w2_ref, b2_ref, g2_ref,
               n2_ref, out_ref):
    h = x_ref[...] @ w1_ref[...] + b1_ref[...]
    h = _gelu(_ln(h, g1_ref[...], n1_ref[...]))
    h = h @ w2_ref[...] + b2_ref[...]
    h = _ln(h, g2_ref[...], n2_ref[...])
    out_ref[0, ...] = h[:, :128]
    out_ref[1, ...] = h[:, 128:]


def _proj(x, p, bm):
    n = x.shape[0]
    args = (x, p['W1'], p['b1'].reshape(1, H), p['g1'].reshape(1, H),
            p['bn1'].reshape(1, H), p['W2'], p['b2'].reshape(1, H),
            p['g2'].reshape(1, H), p['bn2'].reshape(1, H))
    in_specs = [pl.BlockSpec((bm, D), lambda i: (i, 0))] + \
               [_full2d(a) for a in args[1:]]
    return pl.pallas_call(
        _proj_body,
        grid=(n // bm,),
        in_specs=in_specs,
        out_specs=pl.BlockSpec((2, bm, 128), lambda i: (0, i, 0)),
        out_shape=jax.ShapeDtypeStruct((2, n, 128), jnp.float32),
        interpret=_INTERPRET,
    )(*args)


def _combine_body(k, halves_out, *refs):
    h_ref = refs[0]
    s_refs = refs[1:1 + k]
    c_refs = refs[1 + k:1 + 2 * k]
    wl_refs = refs[1 + 2 * k:1 + 3 * k]
    wr_refs = refs[1 + 3 * k:1 + 4 * k]
    bl_ref, g_ref, b_ref = refs[1 + 4 * k:1 + 4 * k + 3]
    out_ref = refs[-1]
    h = jnp.concatenate([h_ref[0], h_ref[1]], axis=-1)
    o = jnp.zeros_like(h)
    for s_ref, c_ref, wl_ref in zip(s_refs, c_refs, wl_refs):
        s = jnp.concatenate([s_ref[0], s_ref[1]], axis=-1)
        cnt = c_ref[0, :, 0:1] + c_ref[1, :, 0:1]
        mean = s / jnp.maximum(cnt, 1.0)
        o = o + mean @ wl_ref[...]
    wr = wr_refs[0][...]
    for r in wr_refs[1:]:
        wr = wr + r[...]
    o = o + h @ wr + bl_ref[...]
    res = _ln(_gelu(o) + h, g_ref[...], b_ref[...])
    if halves_out:
        out_ref[0, ...] = res[:, :128]
        out_ref[1, ...] = res[:, 128:]
    else:
        out_ref[...] = res


def _combine(h2, sums, cnts, wls, wrs, bls, g, b, bm, halves_out):
    k = len(sums)
    n = h2.shape[1]
    bl = bls[0]
    for x in bls[1:]:
        bl = bl + x
    args = ([h2] + list(sums) + list(cnts) + list(wls) + list(wrs)
            + [bl.reshape(1, H), g.reshape(1, H), b.reshape(1, H)])
    in_specs = ([pl.BlockSpec((2, bm, 128), lambda i: (0, i, 0))]
                + [pl.BlockSpec((2, bm, 128), lambda i: (0, i, 0))] * k
                + [pl.BlockSpec((2, bm, 16), lambda i: (0, i, 0))] * k
                + [_full2d(a) for a in args[1 + 2 * k:]])
    if halves_out:
        out_spec = pl.BlockSpec((2, bm, 128), lambda i: (0, i, 0))
        out_shape = jax.ShapeDtypeStruct((2, n, 128), jnp.float32)
    else:
        out_spec = pl.BlockSpec((bm, H), lambda i: (i, 0))
        out_shape = jax.ShapeDtypeStruct((n, H), jnp.float32)
    return pl.pallas_call(
        functools.partial(_combine_body, k, halves_out),
        grid=(n // bm,),
        in_specs=in_specs,
        out_specs=out_spec,
        out_shape=out_shape,
        interpret=_INTERPRET,
    )(*args)


def _head_body(hc_ref, ctx_ref, w1c_ref, w1x_ref, b1_ref, w2_ref, b2_ref,
               out_ref):
    z = _gelu(hc_ref[...] @ w1c_ref[...] + ctx_ref[...] @ w1x_ref[...]
              + b1_ref[...])
    out_ref[...] = z @ w2_ref[...] + b2_ref[0, 0]


def _head(hc, ctx, p, bm):
    n = hc.shape[0]
    w2p = jnp.pad(p['W2'], ((0, 0), (0, 127)))
    args = (hc, ctx, p['W1'][:H], p['W1'][H:], p['b1'].reshape(1, H), w2p,
            p['b2'].reshape(1, 1))
    in_specs = [pl.BlockSpec((bm, H), lambda i: (i, 0)),
                pl.BlockSpec((bm, H), lambda i: (i, 0))] + \
               [_full2d(a) for a in args[2:]]
    out = pl.pallas_call(
        _head_body,
        grid=(n // bm,),
        in_specs=in_specs,
        out_specs=pl.BlockSpec((bm, 128), lambda i: (i, 0)),
        out_shape=jax.ShapeDtypeStruct((n, 128), jnp.float32),
        interpret=_INTERPRET,
    )(*args)
    return out[:, 0]


def _seg_sums(h2_by_type, eis, dst_n):
    out = {}
    for et, ei in eis.items():
        src, dst = et.split('2')
        h2 = h2_by_type[src]
        n = dst_n[dst]
        x = jnp.concatenate([h2[0], h2[1]], axis=-1)
        s = jax.ops.segment_sum(x[ei[0]], ei[1], num_segments=n)
        out[et] = jnp.stack([s[:, :128], s[:, 128:]])
    return out


def _seg_counts(eis, dst_n):
    out = {}
    for et, ei in eis.items():
        dst = et.split('2')[1]
        n = dst_n[dst]
        c = jax.ops.segment_sum(jnp.ones((ei.shape[1],), jnp.float32), ei[1],
                                num_segments=n)
        c16 = jnp.broadcast_to(c[:, None], (n, 16))
        out[et] = jnp.stack([c16, jnp.zeros_like(c16)])
    return out


def kernel(x_global, x_lesion, x_cause, global_txt, text_mask, ei_g2l, ei_l2g,
           ei_l2l, ei_l2c, ei_c2l, ei_g2c, ei_c2g, cause_batch, params):
    B = x_global.shape[0]
    NL = x_lesion.shape[0]
    NC = x_cause.shape[0]
    eis = {'g2l': ei_g2l, 'l2g': ei_l2g, 'l2l': ei_l2l, 'l2c': ei_l2c,
           'c2l': ei_c2l, 'g2c': ei_g2c, 'c2g': ei_c2g}
    eis = {k: v.astype(jnp.int32) for k, v in eis.items()}
    dst_n = {'g': B, 'l': NL, 'c': NC}

    hg2 = _fuse_proj(x_global, global_txt, text_mask, params['gate'],
                     params['fuse'], params['proj_global'], bm=256)
    hl2 = _proj(x_lesion, params['proj_lesion'], bm=512)
    hc2 = _proj(x_cause, params['proj_cause'], bm=512)

    cnts = _seg_counts(eis, dst_n)

    h2 = {'g': hg2, 'l': hl2, 'c': hc2}
    for bi, bp in enumerate(params['blocks']):
        sums = _seg_sums(h2, eis, dst_n)
        last = bi == len(params['blocks']) - 1
        new = {}
        for dst, ets, bmv in (('l', ('g2l', 'l2l', 'c2l'), 512),
                              ('g', ('l2g', 'c2g'), 256),
                              ('c', ('l2c', 'g2c'), 512)):
            new[dst] = _combine(
                h2[dst], [sums[e] for e in ets], [cnts[e] for e in ets],
                [bp[e]['Wl'] for e in ets], [bp[e]['Wr'] for e in ets],
                [bp[e]['bl'] for e in ets],
                bp['n' + dst + '_g'], bp['n' + dst + '_b'],
                bm=bmv, halves_out=not last)
        h2 = new

    hg, hl, hc = h2['g'], h2['l'], h2['c']
    ctx = hg[cause_batch]
    scores = _head(hc, ctx, params['head'], bm=512)
    return scores, hc, hg, hl


# speedup vs baseline: 2.8216x; 2.6461x over previous
import functools

import jax
import jax.numpy as jnp
from jax.experimental import pallas as pl
from jax.experimental.pallas import tpu as pltpu
from jax.experimental.pallas import tpu_sc as plsc

D = 512
H = 256
_INTERPRET = False


def _ln(x, g, b, eps=1e-5):
    m = x.mean(-1, keepdims=True)
    v = ((x - m) ** 2).mean(-1, keepdims=True)
    return (x - m) / jnp.sqrt(v + eps) * g + b


def _gelu(x):
    return x * 0.5 * (1.0 + jax.lax.erf(x * (2.0 ** -0.5)))


def _full2d(a):
    return pl.BlockSpec(a.shape, lambda i: (0, 0))


def _fuse_proj_body(xg_ref, gt_ref, tm_ref, gate_ref, wfg_ref, wft_ref, bf_ref,
                    w1_ref, b1_ref, g1_ref, n1_ref, w2_ref, b2_ref, g2_ref,
                    n2_ref, out_ref):
    xg = xg_ref[...]
    fused = _gelu(xg @ wfg_ref[...] + gt_ref[...] @ wft_ref[...] + bf_ref[...])
    g = xg + gate_ref[0, 0] * (tm_ref[...] * fused)
    h = g @ w1_ref[...] + b1_ref[...]
    h = _gelu(_ln(h, g1_ref[...], n1_ref[...]))
    h = h @ w2_ref[...] + b2_ref[...]
    h = _ln(h, g2_ref[...], n2_ref[...])
    out_ref[0, ...] = h[:, :128]
    out_ref[1, ...] = h[:, 128:]


def _fuse_proj(xg, gt, tm, gate, fuse_p, proj_p, bm):
    n = xg.shape[0]
    wfg = fuse_p['W'][:D]
    wft = fuse_p['W'][D:]
    args = (xg, gt, tm.reshape(n, 1), gate.reshape(1, 1), wfg, wft,
            fuse_p['b'].reshape(1, D),
            proj_p['W1'], proj_p['b1'].reshape(1, H),
            proj_p['g1'].reshape(1, H), proj_p['bn1'].reshape(1, H),
            proj_p['W2'], proj_p['b2'].reshape(1, H),
            proj_p['g2'].reshape(1, H), proj_p['bn2'].reshape(1, H))
    in_specs = [
        pl.BlockSpec((bm, D), lambda i: (i, 0)),
        pl.BlockSpec((bm, D), lambda i: (i, 0)),
        pl.BlockSpec((bm, 1), lambda i: (i, 0)),
        pl.BlockSpec((1, 1), lambda i: (0, 0)),
    ] + [_full2d(a) for a in args[4:]]
    return pl.pallas_call(
        _fuse_proj_body,
        grid=(n // bm,),
        in_specs=in_specs,
        out_specs=pl.BlockSpec((2, bm, 128), lambda i: (0, i, 0)),
        out_shape=jax.ShapeDtypeStruct((2, n, 128), jnp.float32),
        interpret=_INTERPRET,
    )(*args)


def _proj_body(x_ref, w1_ref, b1_ref, g1_ref, n1_ref, w2_ref, b2_ref, g2_ref,
               n2_ref, out_ref):
    h = x_ref[...] @ w1_ref[...] + b1_ref[...]
    h = _gelu(_ln(h, g1_ref[...], n1_ref[...]))
    h = h @ w2_ref[...] + b2_ref[...]
    h = _ln(h, g2_ref[...], n2_ref[...])
    out_ref[0, ...] = h[:, :128]
    out_ref[1, ...] = h[:, 128:]


def _proj(x, p, bm):
    n = x.shape[0]
    args = (x, p['W1'], p['b1'].reshape(1, H), p['g1'].reshape(1, H),
            p['bn1'].reshape(1, H), p['W2'], p['b2'].reshape(1, H),
            p['g2'].reshape(1, H), p['bn2'].reshape(1, H))
    in_specs = [pl.BlockSpec((bm, D), lambda i: (i, 0))] + \
               [_full2d(a) for a in args[1:]]
    return pl.pallas_call(
        _proj_body,
        grid=(n // bm,),
        in_specs=in_specs,
        out_specs=pl.BlockSpec((2, bm, 128), lambda i: (0, i, 0)),
        out_shape=jax.ShapeDtypeStruct((2, n, 128), jnp.float32),
        interpret=_INTERPRET,
    )(*args)


def _combine_body(k, halves_out, *refs):
    h_ref = refs[0]
    s_refs = refs[1:1 + k]
    c_refs = refs[1 + k:1 + 2 * k]
    wl_refs = refs[1 + 2 * k:1 + 3 * k]
    wr_refs = refs[1 + 3 * k:1 + 4 * k]
    bl_ref, g_ref, b_ref = refs[1 + 4 * k:1 + 4 * k + 3]
    out_ref = refs[-1]
    h = jnp.concatenate([h_ref[0], h_ref[1]], axis=-1)
    o = jnp.zeros_like(h)
    for s_ref, c_ref, wl_ref in zip(s_refs, c_refs, wl_refs):
        s = jnp.concatenate([s_ref[0], s_ref[1]], axis=-1)
        cnt = c_ref[0, :, 0:1] + c_ref[1, :, 0:1]
        mean = s / jnp.maximum(cnt, 1.0)
        o = o + mean @ wl_ref[...]
    wr = wr_refs[0][...]
    for r in wr_refs[1:]:
        wr = wr + r[...]
    o = o + h @ wr + bl_ref[...]
    res = _ln(_gelu(o) + h, g_ref[...], b_ref[...])
    if halves_out:
        out_ref[0, ...] = res[:, :128]
        out_ref[1, ...] = res[:, 128:]
    else:
        out_ref[...] = res


def _combine(h2, sums, cnts, wls, wrs, bls, g, b, bm, halves_out):
    k = len(sums)
    n = h2.shape[1]
    bl = bls[0]
    for x in bls[1:]:
        bl = bl + x
    args = ([h2] + list(sums) + list(cnts) + list(wls) + list(wrs)
            + [bl.reshape(1, H), g.reshape(1, H), b.reshape(1, H)])
    in_specs = ([pl.BlockSpec((2, bm, 128), lambda i: (0, i, 0))]
                + [pl.BlockSpec((2, bm, 128), lambda i: (0, i, 0))] * k
                + [pl.BlockSpec((2, bm, 128), lambda i: (0, i, 0))] * k
                + [_full2d(a) for a in args[1 + 2 * k:]])
    if halves_out:
        out_spec = pl.BlockSpec((2, bm, 128), lambda i: (0, i, 0))
        out_shape = jax.ShapeDtypeStruct((2, n, 128), jnp.float32)
    else:
        out_spec = pl.BlockSpec((bm, H), lambda i: (i, 0))
        out_shape = jax.ShapeDtypeStruct((n, H), jnp.float32)
    return pl.pallas_call(
        functools.partial(_combine_body, k, halves_out),
        grid=(n // bm,),
        in_specs=in_specs,
        out_specs=out_spec,
        out_shape=out_shape,
        interpret=_INTERPRET,
    )(*args)


def _head_body(hc_ref, ctx_ref, w1c_ref, w1x_ref, b1_ref, w2_ref, b2_ref,
               out_ref):
    z = _gelu(hc_ref[...] @ w1c_ref[...] + ctx_ref[...] @ w1x_ref[...]
              + b1_ref[...])
    out_ref[...] = z @ w2_ref[...] + b2_ref[0, 0]


def _head(hc, ctx, p, bm):
    n = hc.shape[0]
    w2p = jnp.pad(p['W2'], ((0, 0), (0, 127)))
    args = (hc, ctx, p['W1'][:H], p['W1'][H:], p['b1'].reshape(1, H), w2p,
            p['b2'].reshape(1, 1))
    in_specs = [pl.BlockSpec((bm, H), lambda i: (i, 0)),
                pl.BlockSpec((bm, H), lambda i: (i, 0))] + \
               [_full2d(a) for a in args[2:]]
    out = pl.pallas_call(
        _head_body,
        grid=(n // bm,),
        in_specs=in_specs,
        out_specs=pl.BlockSpec((bm, 128), lambda i: (i, 0)),
        out_shape=jax.ShapeDtypeStruct((n, 128), jnp.float32),
        interpret=_INTERPRET,
    )(*args)
    return out[:, 0]


_CHUNK = 128
_WCH = 64


def _sc_counts(dsts, dst_sizes):
    mesh = plsc.VectorSubcoreMesh(core_axis_name="c", subcore_axis_name="s")
    nt = len(dsts)
    max_n = max(dst_sizes)
    out_type = [jax.ShapeDtypeStruct((2, n, 128), jnp.float32)
                for n in dst_sizes]

    @functools.partial(
        pl.kernel, mesh=mesh, out_type=out_type,
        scratch_types=[
            pltpu.VMEM((1, _CHUNK), jnp.int32),
            pltpu.VMEM((_CHUNK, 128), jnp.float32),
            pltpu.VMEM((_WCH, 128), jnp.float32),
            pltpu.VMEM_SHARED((max_n, 128), jnp.float32),
        ],
    )
    def k(*refs):
        dst_refs = refs[:nt]
        ones_hbm = refs[nt]
        out_refs = refs[nt + 1:nt + 1 + nt]
        dstb, ones, zb, acc = refs[nt + 1 + nt:]
        c = jax.lax.axis_index("c")
        s = jax.lax.axis_index("s")
        w = c * 16 + s
        pltpu.sync_copy(ones_hbm, ones)

        def fill_z(i, _):
            for j in range(8):
                zb[i, pl.ds(j * 16, 16)] = jnp.zeros((16,), jnp.float32)
            return 0
        jax.lax.fori_loop(0, _WCH, fill_z, 0)

        for dref, oref in zip(dst_refs, out_refs):
            e = dref.shape[0]
            n = oref.shape[1]
            rows = n // 16
            ew = e // 32
            for j in range(rows // _WCH):
                pltpu.sync_copy(zb, acc.at[pl.ds(s * rows + j * _WCH,
                                                 _WCH)])
            plsc.subcore_barrier()

            def body(i, _):
                off = w * ew + i * _CHUNK
                pltpu.sync_copy(dref.at[pl.ds(off, _CHUNK)], dstb.at[0])
                pltpu.sync_copy(ones, acc.at[dstb.at[0]], add=True)
                return 0
            jax.lax.fori_loop(0, ew // _CHUNK, body, 0)
            plsc.subcore_barrier()
            for j in range(rows // _WCH):
                r0 = s * rows + j * _WCH
                pltpu.sync_copy(acc.at[pl.ds(r0, _WCH)],
                                oref.at[c, pl.ds(r0, _WCH)])
            plsc.subcore_barrier()

    return k(*dsts, jnp.ones((_CHUNK, 128), jnp.float32))


def _sc_block_sums(h2f, ets, srcs, dsts, src_n, dst_n):
    mesh = plsc.VectorSubcoreMesh(core_axis_name="c", subcore_axis_name="s")
    nt = len(ets)
    max_n = max(dst_n[et.split('2')[1]] for et in ets)
    out_type = [jax.ShapeDtypeStruct((2, dst_n[et.split('2')[1]], 128),
                                     jnp.float32) for et in ets]
    tbls = [h2f['g'], h2f['l'], h2f['c']]
    tbl_of = {'g': 0, 'l': 1, 'c': 2}

    @functools.partial(
        pl.kernel, mesh=mesh, out_type=out_type,
        scratch_types=[
            pltpu.VMEM((_CHUNK,), jnp.int32),
            pltpu.VMEM((_CHUNK,), jnp.int32),
            pltpu.VMEM((1, _CHUNK), jnp.int32),
            pltpu.VMEM((_CHUNK, 128), jnp.float32),
            pltpu.VMEM((_WCH, 128), jnp.float32),
            pltpu.VMEM_SHARED((max_n, 128), jnp.float32),
            pltpu.SemaphoreType.DMA,
        ],
    )
    def k(*refs):
        tbl_refs = refs[:3]
        src_refs = refs[3:3 + nt]
        dst_refs = refs[3 + nt:3 + 2 * nt]
        out_refs = refs[3 + 2 * nt:3 + 3 * nt]
        srcb, srca, dstb, rowsb, zb, acc = refs[3 + 3 * nt:3 + 3 * nt + 6]
        gsem = refs[-1]
        c = jax.lax.axis_index("c")
        s = jax.lax.axis_index("s")

        def fill_z(i, _):
            for j in range(8):
                zb[i, pl.ds(j * 16, 16)] = jnp.zeros((16,), jnp.float32)
            return 0
        jax.lax.fori_loop(0, _WCH, fill_z, 0)

        if True:
            for t, et in enumerate(ets):
                skey, dkey = et.split('2')
                tref = tbl_refs[tbl_of[skey]]
                sref, dref, oref = src_refs[t], dst_refs[t], out_refs[t]
                nsrc = tref.shape[0] // 2
                e = sref.shape[0]
                n = oref.shape[1]
                rows = n // 16
                et_per_tile = e // 16
                for j in range(rows // _WCH):
                    pltpu.sync_copy(zb, acc.at[pl.ds(s * rows + j * _WCH,
                                                     _WCH)])
                plsc.subcore_barrier()

                def body(i, _):
                    off = s * et_per_tile + i * _CHUNK
                    pltpu.sync_copy(sref.at[pl.ds(off, _CHUNK)], srcb)
                    pltpu.sync_copy(dref.at[pl.ds(off, _CHUNK)], dstb.at[0])
                    for j in range(_CHUNK // 16):
                        srca[pl.ds(j * 16, 16)] = (
                            srcb[pl.ds(j * 16, 16)] + c * nsrc)
                    pltpu.async_copy(tref.at[srca], rowsb, gsem).wait()
                    pltpu.sync_copy(rowsb, acc.at[dstb.at[0]], add=True)
                    return 0
                jax.lax.fori_loop(0, et_per_tile // _CHUNK, body, 0)
                plsc.subcore_barrier()
                for j in range(rows // _WCH):
                    r0 = s * rows + j * _WCH
                    pltpu.sync_copy(acc.at[pl.ds(r0, _WCH)],
                                    oref.at[c, pl.ds(r0, _WCH)])
                plsc.subcore_barrier()

    return k(*tbls, *srcs, *dsts)


def _sc_row_gather(tbl, idx):
    q = idx.shape[0]
    mesh = plsc.VectorSubcoreMesh(core_axis_name="c", subcore_axis_name="s")
    qw = q // 32

    @functools.partial(
        pl.kernel, mesh=mesh,
        out_type=jax.ShapeDtypeStruct((q, 256), jnp.float32),
        scratch_types=[
            pltpu.VMEM((_CHUNK,), jnp.int32),
            pltpu.VMEM((_CHUNK, 256), jnp.float32),
            pltpu.SemaphoreType.DMA,
        ],
    )
    def k(tbl_ref, idx_ref, out_ref, idxb, rowsb, sem):
        c = jax.lax.axis_index("c")
        s = jax.lax.axis_index("s")
        w = c * 16 + s

        def body(i, _):
            off = w * qw + i * _CHUNK
            pltpu.sync_copy(idx_ref.at[pl.ds(off, _CHUNK)], idxb)
            pltpu.async_copy(tbl_ref.at[idxb], rowsb, sem).wait()
            pltpu.sync_copy(rowsb, out_ref.at[pl.ds(off, _CHUNK)])
            return 0
        jax.lax.fori_loop(0, qw // _CHUNK, body, 0)

    return k(tbl, idx)


def kernel(x_global, x_lesion, x_cause, global_txt, text_mask, ei_g2l, ei_l2g,
           ei_l2l, ei_l2c, ei_c2l, ei_g2c, ei_c2g, cause_batch, params):
    B = x_global.shape[0]
    NL = x_lesion.shape[0]
    NC = x_cause.shape[0]
    eis = {'g2l': ei_g2l, 'l2g': ei_l2g, 'l2l': ei_l2l, 'l2c': ei_l2c,
           'c2l': ei_c2l, 'g2c': ei_g2c, 'c2g': ei_c2g}
    eis = {k: v.astype(jnp.int32) for k, v in eis.items()}
    dst_n = {'g': B, 'l': NL, 'c': NC}

    et_order = list(eis.keys())

    hg2 = _fuse_proj(x_global, global_txt, text_mask, params['gate'],
                     params['fuse'], params['proj_global'], bm=256)
    hl2 = _proj(x_lesion, params['proj_lesion'], bm=512)
    hc2 = _proj(x_cause, params['proj_cause'], bm=512)

    cnt_list = _sc_counts([eis[e][1] for e in et_order],
                          [dst_n[e.split('2')[1]] for e in et_order])
    cnts = dict(zip(et_order, cnt_list))

    h2 = {'g': hg2, 'l': hl2, 'c': hc2}
    for bi, bp in enumerate(params['blocks']):
        h2f = {k: v.reshape(2 * v.shape[1], 128) for k, v in h2.items()}
        sum_list = _sc_block_sums(h2f, et_order,
                                  [eis[e][0] for e in et_order],
                                  [eis[e][1] for e in et_order],
                                  {k: v.shape[1] for k, v in h2.items()},
                                  dst_n)
        sums = dict(zip(et_order, sum_list))
        last = bi == len(params['blocks']) - 1
        new = {}
        for dst, ets, bmv in (('l', ('g2l', 'l2l', 'c2l'), 512),
                              ('g', ('l2g', 'c2g'), 256),
                              ('c', ('l2c', 'g2c'), 512)):
            new[dst] = _combine(
                h2[dst], [sums[e] for e in ets], [cnts[e] for e in ets],
                [bp[e]['Wl'] for e in ets], [bp[e]['Wr'] for e in ets],
                [bp[e]['bl'] for e in ets],
                bp['n' + dst + '_g'], bp['n' + dst + '_b'],
                bm=bmv, halves_out=not last)
        h2 = new

    hg, hl, hc = h2['g'], h2['l'], h2['c']
    ctx = _sc_row_gather(hg, cause_batch.astype(jnp.int32))
    scores = _head(hc, ctx, params['head'], bm=512)
    return scores, hc, hg, hl
